# baseline (speedup 1.0000x reference)
import functools

import jax
import jax.numpy as jnp
from jax import lax
from jax.experimental import pallas as pl
from jax.experimental.pallas import tpu as pltpu
from jax.experimental.pallas import tpu_sc as plsc

N = 10000
E = 320000
H = 64

NC = 2
NS = 16
GRP = 128

N_PAD = 10240
ROWS_PER_TILE = N_PAD // NS
E_PER_TILE_GRPS = 160
E_PAD = NS * E_PER_TILE_GRPS * GRP

BLK = 1024


_sc_mesh = plsc.VectorSubcoreMesh(core_axis_name="c", subcore_axis_name="s")


@functools.partial(
    pl.kernel,
    out_type=(jax.ShapeDtypeStruct((N_PAD, H), jnp.float32),
              jax.ShapeDtypeStruct((N_PAD, H), jnp.float32)),
    mesh=_sc_mesh,
    compiler_params=pltpu.CompilerParams(use_tc_tiling_on_sc=False),
    scratch_types=[
        pltpu.VMEM_SHARED((N_PAD, H), jnp.float32),
        pltpu.VMEM((E_PER_TILE_GRPS, GRP), jnp.int32),
        pltpu.VMEM((E_PER_TILE_GRPS, GRP), jnp.int32),
        pltpu.VMEM((GRP, H), jnp.float32),
        pltpu.VMEM((GRP, H), jnp.float32),
        pltpu.VMEM((GRP, H), jnp.float32),
        pltpu.VMEM((GRP, H), jnp.float32),
        pltpu.VMEM((GRP, H), jnp.float32),
        pltpu.SemaphoreType.DMA,
        pltpu.SemaphoreType.DMA,
        pltpu.SemaphoreType.DMA,
        pltpu.SemaphoreType.DMA,
        pltpu.SemaphoreType.DMA,
    ],
)
def _sc_dual_segment_sum(yf, yb, src3, dst3, zeros, out_f, out_b,
                         acc, idx_in, idx_out, r0, r1, r2, r3, r4,
                         g0, g1, g2, g3, g4):
    bufs = (r0, r1, r2, r3, r4)
    semg = (g0, g1, g2, g3, g4)
    c = lax.axis_index("c")
    s = lax.axis_index("s")

    pltpu.sync_copy(zeros, acc.at[pl.ds(s * ROWS_PER_TILE, ROWS_PER_TILE)])

    @pl.when(c == 0)
    def _():
        pltpu.sync_copy(src3.at[s], idx_in)
        pltpu.sync_copy(dst3.at[s], idx_out)

    @pl.when(c == 1)
    def _():
        pltpu.sync_copy(dst3.at[s], idx_in)
        pltpu.sync_copy(src3.at[s], idx_out)

    plsc.subcore_barrier()

    def sweep(table):
        nb = 5
        for b in range(nb - 1):
            pltpu.async_copy(table.at[idx_in.at[b]], bufs[b], semg[b])

        def body(i, carry):
            base = nb * i
            for p in range(nb):
                j = base + p
                q = (p + nb - 1) % nb
                pltpu.async_copy(table.at[idx_in.at[j + nb - 1]], bufs[q],
                                 semg[q])
                pltpu.make_async_copy(table.at[idx_in.at[j]], bufs[p],
                                      semg[p]).wait()
                pltpu.sync_copy(bufs[p], acc.at[idx_out.at[j]], add=True)
            return carry

        lax.fori_loop(0, E_PER_TILE_GRPS // nb - 1, body, 0)

        last = E_PER_TILE_GRPS - nb
        pltpu.async_copy(table.at[idx_in.at[last + nb - 1]],
                         bufs[nb - 1], semg[nb - 1])
        for p in range(nb):
            j = last + p
            pltpu.make_async_copy(table.at[idx_in.at[j]], bufs[p],
                                  semg[p]).wait()
            pltpu.sync_copy(bufs[p], acc.at[idx_out.at[j]], add=True)

    @pl.when(c == 0)
    def _():
        sweep(yf)

    @pl.when(c == 1)
    def _():
        sweep(yb)

    plsc.subcore_barrier()

    @pl.when(c == 0)
    def _():
        pltpu.sync_copy(acc.at[pl.ds(s * ROWS_PER_TILE, ROWS_PER_TILE)],
                        out_f.at[pl.ds(s * ROWS_PER_TILE, ROWS_PER_TILE)])

    @pl.when(c == 1)
    def _():
        pltpu.sync_copy(acc.at[pl.ds(s * ROWS_PER_TILE, ROWS_PER_TILE)],
                        out_b.at[pl.ds(s * ROWS_PER_TILE, ROWS_PER_TILE)])


def _full(shape):
    return pl.BlockSpec(shape, lambda i: (0,) * len(shape))


def _rows(shape):
    return pl.BlockSpec(shape, lambda i: (i,) + (0,) * (len(shape) - 1))


def _stage_a_body(x, wnf, wnb, wrf, wrb, b1f, b1b, wp, bp,
                  y1f, y1b, r1f, r1b, proj):
    xb = x[...]
    y1f[...] = jnp.dot(xb, wnf[...], preferred_element_type=jnp.float32)
    y1b[...] = jnp.dot(xb, wnb[...], preferred_element_type=jnp.float32)
    r1f[...] = jnp.dot(xb, wrf[...], preferred_element_type=jnp.float32) + b1f[...]
    r1b[...] = jnp.dot(xb, wrb[...], preferred_element_type=jnp.float32) + b1b[...]
    proj[...] = jnp.dot(xb, wp[...], preferred_element_type=jnp.float32) + bp[...]


def _stage_a(x_pad, wnf, wnb, wrf, wrb, b1f, b1b, wp, bp):
    n = x_pad.shape[0]
    o64 = jax.ShapeDtypeStruct((n, H), jnp.float32)
    o128 = jax.ShapeDtypeStruct((n, 128), jnp.float32)
    return pl.pallas_call(
        _stage_a_body,
        grid=(n // BLK,),
        in_specs=[_rows((BLK, 128)), _full((128, H)), _full((128, H)),
                  _full((128, H)), _full((128, H)), _full((1, H)),
                  _full((1, H)), _full((128, 128)), _full((1, 128))],
        out_specs=[_rows((BLK, H)), _rows((BLK, H)), _rows((BLK, H)),
                   _rows((BLK, H)), _rows((BLK, 128))],
        out_shape=[o64, o64, o64, o64, o128],
    )(x_pad, wnf, wnb, wrf, wrb, b1f, b1b, wp, bp)


def _stage_b_body(msgf, msgb, r1f, r1b, wfct, wfcb, bfc,
                  wnf, wnb, wrf, wrb, b2f, b2b,
                  y2f, y2b, r2f, r2b):
    hf = jnp.maximum(r1f[...] + msgf[...], 0.0)
    hb = jnp.maximum(r1b[...] + msgb[...], 0.0)
    h = (jnp.dot(hf, wfct[...], preferred_element_type=jnp.float32)
         + jnp.dot(hb, wfcb[...], preferred_element_type=jnp.float32)
         + bfc[...])
    y2f[...] = jnp.dot(h, wnf[...], preferred_element_type=jnp.float32)
    y2b[...] = jnp.dot(h, wnb[...], preferred_element_type=jnp.float32)
    r2f[...] = jnp.dot(h, wrf[...], preferred_element_type=jnp.float32) + b2f[...]
    r2b[...] = jnp.dot(h, wrb[...], preferred_element_type=jnp.float32) + b2b[...]


def _stage_b(msgf, msgb, r1f, r1b, wfct, wfcb, bfc, wnf, wnb, wrf, wrb,
             b2f, b2b):
    n = msgf.shape[0]
    o64 = jax.ShapeDtypeStruct((n, H), jnp.float32)
    w64 = _full((H, H))
    return pl.pallas_call(
        _stage_b_body,
        grid=(n // BLK,),
        in_specs=[_rows((BLK, H))] * 4 + [w64, w64, _full((1, H)),
                                          w64, w64, w64, w64,
                                          _full((1, H)), _full((1, H))],
        out_specs=[_rows((BLK, H))] * 4,
        out_shape=[o64, o64, o64, o64],
    )(msgf, msgb, r1f, r1b, wfct, wfcb, bfc, wnf, wnb, wrf, wrb, b2f, b2b)


def _stage_c_body(msgf, msgb, r2f, r2b, proj, out):
    of = r2f[...] + msgf[...]
    ob = r2b[...] + msgb[...]
    out[...] = jnp.concatenate([of, ob], axis=-1) + proj[...]


def _stage_c(msgf, msgb, r2f, r2b, proj):
    n = msgf.shape[0]
    return pl.pallas_call(
        _stage_c_body,
        grid=(n // BLK,),
        in_specs=[_rows((BLK, H))] * 4 + [_rows((BLK, 128))],
        out_specs=_rows((BLK, 128)),
        out_shape=jax.ShapeDtypeStruct((n, 128), jnp.float32),
    )(msgf, msgb, r2f, r2b, proj)


def kernel(x, edge_index, W1_root_f, W1_nbr_f, b1_f, W1_root_b, W1_nbr_b,
           b1_b, W_fc, b_fc, W2_root_f, W2_nbr_f, b2_f, W2_root_b, W2_nbr_b,
           b2_b, W_proj, b_proj):
    f32 = jnp.float32

    x_pad = jnp.zeros((N_PAD, 128), f32).at[:N].set(x)

    src = edge_index[0]
    dst = edge_index[1]
    pad = N + (jnp.arange(E_PAD - E, dtype=jnp.int32) % (N_PAD - N))
    src3 = jnp.concatenate([src, pad]).reshape(NS, E_PER_TILE_GRPS, GRP)
    dst3 = jnp.concatenate([dst, pad]).reshape(NS, E_PER_TILE_GRPS, GRP)
    zeros = jnp.zeros((ROWS_PER_TILE, H), f32)

    r1 = lambda b: b.reshape(1, -1)

    y1f, y1b, r1f, r1b, proj = _stage_a(
        x_pad, W1_nbr_f, W1_nbr_b, W1_root_f, W1_root_b,
        r1(b1_f), r1(b1_b), W_proj, r1(b_proj))

    msg1f, msg1b = _sc_dual_segment_sum(y1f, y1b, src3, dst3, zeros)

    y2f, y2b, r2f, r2b = _stage_b(
        msg1f, msg1b, r1f, r1b, W_fc[:H], W_fc[H:], r1(b_fc),
        W2_nbr_f, W2_nbr_b, W2_root_f, W2_root_b, r1(b2_f), r1(b2_b))

    msg2f, msg2b = _sc_dual_segment_sum(y2f, y2b, src3, dst3, zeros)

    out_pad = _stage_c(msg2f, msg2b, r2f, r2b, proj)
    return out_pad[:N]


# speedup vs baseline: 1.0136x; 1.0136x over previous
import functools

import jax
import jax.numpy as jnp
from jax import lax
from jax.experimental import pallas as pl
from jax.experimental.pallas import tpu as pltpu
from jax.experimental.pallas import tpu_sc as plsc

N = 10000
E = 320000
H = 64

NC = 2
NS = 16
GRP = 128

N_PAD = 10240
ROWS_PER_TILE = N_PAD // NS
E_PER_TILE_GRPS = 160
E_PAD = NS * E_PER_TILE_GRPS * GRP

BLK = 1024


_sc_mesh = plsc.VectorSubcoreMesh(core_axis_name="c", subcore_axis_name="s")


@functools.partial(
    pl.kernel,
    out_type=(jax.ShapeDtypeStruct((N_PAD, H), jnp.float32),
              jax.ShapeDtypeStruct((N_PAD, H), jnp.float32)),
    mesh=_sc_mesh,
    compiler_params=pltpu.CompilerParams(use_tc_tiling_on_sc=False),
    scratch_types=[
        pltpu.VMEM_SHARED((N_PAD, H), jnp.float32),
        pltpu.VMEM((E_PER_TILE_GRPS, GRP), jnp.int32),
        pltpu.VMEM((E_PER_TILE_GRPS, GRP), jnp.int32),
        pltpu.VMEM((GRP, H), jnp.float32),
        pltpu.VMEM((GRP, H), jnp.float32),
        pltpu.VMEM((GRP, H), jnp.float32),
        pltpu.VMEM((GRP, H), jnp.float32),
        pltpu.VMEM((GRP, H), jnp.float32),
        pltpu.SemaphoreType.DMA,
        pltpu.SemaphoreType.DMA,
        pltpu.SemaphoreType.DMA,
        pltpu.SemaphoreType.DMA,
        pltpu.SemaphoreType.DMA,
    ],
)
def _sc_dual_segment_sum(yf, yb, src3, dst3, zeros, out_f, out_b,
                         acc, idx_in, idx_out, r0, r1, r2, r3, r4,
                         g0, g1, g2, g3, g4):
    bufs = (r0, r1, r2, r3, r4)
    semg = (g0, g1, g2, g3, g4)
    c = lax.axis_index("c")
    s = lax.axis_index("s")

    pltpu.sync_copy(zeros, acc.at[pl.ds(s * ROWS_PER_TILE, ROWS_PER_TILE)])

    @pl.when(c == 0)
    def _():
        pltpu.sync_copy(src3.at[s], idx_in)
        pltpu.sync_copy(dst3.at[s], idx_out)

    @pl.when(c == 1)
    def _():
        pltpu.sync_copy(dst3.at[s], idx_in)
        pltpu.sync_copy(src3.at[s], idx_out)

    plsc.subcore_barrier()

    def sweep(table):
        nb = 5
        for b in range(nb - 1):
            pltpu.async_copy(table.at[idx_in.at[b]], bufs[b], semg[b])

        def body(i, carry):
            base = nb * i
            for p in range(nb):
                j = base + p
                q = (p + nb - 1) % nb
                pltpu.async_copy(table.at[idx_in.at[j + nb - 1]], bufs[q],
                                 semg[q])
                pltpu.make_async_copy(table.at[idx_in.at[j]], bufs[p],
                                      semg[p]).wait()
                pltpu.sync_copy(bufs[p], acc.at[idx_out.at[j]], add=True)
            return carry

        lax.fori_loop(0, E_PER_TILE_GRPS // nb - 1, body, 0)

        last = E_PER_TILE_GRPS - nb
        pltpu.async_copy(table.at[idx_in.at[last + nb - 1]],
                         bufs[nb - 1], semg[nb - 1])
        for p in range(nb):
            j = last + p
            pltpu.make_async_copy(table.at[idx_in.at[j]], bufs[p],
                                  semg[p]).wait()
            pltpu.sync_copy(bufs[p], acc.at[idx_out.at[j]], add=True)

    @pl.when(c == 0)
    def _():
        sweep(yf)

    @pl.when(c == 1)
    def _():
        sweep(yb)

    plsc.subcore_barrier()

    @pl.when(c == 0)
    def _():
        pltpu.sync_copy(acc.at[pl.ds(s * ROWS_PER_TILE, ROWS_PER_TILE)],
                        out_f.at[pl.ds(s * ROWS_PER_TILE, ROWS_PER_TILE)])

    @pl.when(c == 1)
    def _():
        pltpu.sync_copy(acc.at[pl.ds(s * ROWS_PER_TILE, ROWS_PER_TILE)],
                        out_b.at[pl.ds(s * ROWS_PER_TILE, ROWS_PER_TILE)])


def _full(shape):
    return pl.BlockSpec(shape, lambda i: (0,) * len(shape))


def _rows(shape):
    return pl.BlockSpec(shape, lambda i: (i,) + (0,) * (len(shape) - 1))


def _stage_a_body(x, wnf, wnb, wrf, wrb, b1f, b1b, wp, bp,
                  y1f, y1b, r1f, r1b, proj):
    xb = x[...]
    y1f[...] = jnp.dot(xb, wnf[...], preferred_element_type=jnp.float32)
    y1b[...] = jnp.dot(xb, wnb[...], preferred_element_type=jnp.float32)
    r1f[...] = jnp.dot(xb, wrf[...], preferred_element_type=jnp.float32) + b1f[...]
    r1b[...] = jnp.dot(xb, wrb[...], preferred_element_type=jnp.float32) + b1b[...]
    proj[...] = jnp.dot(xb, wp[...], preferred_element_type=jnp.float32) + bp[...]


def _stage_a(x, wnf, wnb, wrf, wrb, b1f, b1b, wp, bp):
    o64 = jax.ShapeDtypeStruct((N_PAD, H), jnp.float32)
    o128 = jax.ShapeDtypeStruct((N_PAD, 128), jnp.float32)
    return pl.pallas_call(
        _stage_a_body,
        grid=(N_PAD // BLK,),
        in_specs=[_rows((BLK, 128)), _full((128, H)), _full((128, H)),
                  _full((128, H)), _full((128, H)), _full((1, H)),
                  _full((1, H)), _full((128, 128)), _full((1, 128))],
        out_specs=[_rows((BLK, H)), _rows((BLK, H)), _rows((BLK, H)),
                   _rows((BLK, H)), _rows((BLK, 128))],
        out_shape=[o64, o64, o64, o64, o128],
    )(x, wnf, wnb, wrf, wrb, b1f, b1b, wp, bp)


def _stage_b_body(msgf, msgb, r1f, r1b, wfct, wfcb, bfc,
                  wnf, wnb, wrf, wrb, b2f, b2b,
                  y2f, y2b, r2f, r2b):
    hf = jnp.maximum(r1f[...] + msgf[...], 0.0)
    hb = jnp.maximum(r1b[...] + msgb[...], 0.0)
    h = (jnp.dot(hf, wfct[...], preferred_element_type=jnp.float32)
         + jnp.dot(hb, wfcb[...], preferred_element_type=jnp.float32)
         + bfc[...])
    y2f[...] = jnp.dot(h, wnf[...], preferred_element_type=jnp.float32)
    y2b[...] = jnp.dot(h, wnb[...], preferred_element_type=jnp.float32)
    r2f[...] = jnp.dot(h, wrf[...], preferred_element_type=jnp.float32) + b2f[...]
    r2b[...] = jnp.dot(h, wrb[...], preferred_element_type=jnp.float32) + b2b[...]


def _stage_b(msgf, msgb, r1f, r1b, wfct, wfcb, bfc, wnf, wnb, wrf, wrb,
             b2f, b2b):
    n = msgf.shape[0]
    o64 = jax.ShapeDtypeStruct((n, H), jnp.float32)
    w64 = _full((H, H))
    return pl.pallas_call(
        _stage_b_body,
        grid=(n // BLK,),
        in_specs=[_rows((BLK, H))] * 4 + [w64, w64, _full((1, H)),
                                          w64, w64, w64, w64,
                                          _full((1, H)), _full((1, H))],
        out_specs=[_rows((BLK, H))] * 4,
        out_shape=[o64, o64, o64, o64],
    )(msgf, msgb, r1f, r1b, wfct, wfcb, bfc, wnf, wnb, wrf, wrb, b2f, b2b)


def _stage_c_body(msgf, msgb, r2f, r2b, proj, out):
    of = r2f[...] + msgf[...]
    ob = r2b[...] + msgb[...]
    out[...] = jnp.concatenate([of, ob], axis=-1) + proj[...]


def _stage_c(msgf, msgb, r2f, r2b, proj):
    blk = 1000
    return pl.pallas_call(
        _stage_c_body,
        grid=(N // blk,),
        in_specs=[_rows((blk, H))] * 4 + [_rows((blk, 128))],
        out_specs=_rows((blk, 128)),
        out_shape=jax.ShapeDtypeStruct((N, 128), jnp.float32),
    )(msgf, msgb, r2f, r2b, proj)


def kernel(x, edge_index, W1_root_f, W1_nbr_f, b1_f, W1_root_b, W1_nbr_b,
           b1_b, W_fc, b_fc, W2_root_f, W2_nbr_f, b2_f, W2_root_b, W2_nbr_b,
           b2_b, W_proj, b_proj):
    f32 = jnp.float32

    src = edge_index[0]
    dst = edge_index[1]
    pad = N + (jnp.arange(E_PAD - E, dtype=jnp.int32) % (N_PAD - N))
    src3 = jnp.concatenate([src, pad]).reshape(NS, E_PER_TILE_GRPS, GRP)
    dst3 = jnp.concatenate([dst, pad]).reshape(NS, E_PER_TILE_GRPS, GRP)
    zeros = jnp.zeros((ROWS_PER_TILE, H), f32)

    r1 = lambda b: b.reshape(1, -1)

    y1f, y1b, r1f, r1b, proj = _stage_a(
        x, W1_nbr_f, W1_nbr_b, W1_root_f, W1_root_b,
        r1(b1_f), r1(b1_b), W_proj, r1(b_proj))

    msg1f, msg1b = _sc_dual_segment_sum(y1f, y1b, src3, dst3, zeros)

    y2f, y2b, r2f, r2b = _stage_b(
        msg1f, msg1b, r1f, r1b, W_fc[:H], W_fc[H:], r1(b_fc),
        W2_nbr_f, W2_nbr_b, W2_root_f, W2_root_b, r1(b2_f), r1(b2_b))

    msg2f, msg2b = _sc_dual_segment_sum(y2f, y2b, src3, dst3, zeros)

    return _stage_c(msg2f, msg2b, r2f, r2b, proj)


# speedup vs baseline: 1.0270x; 1.0132x over previous
import functools

import jax
import jax.numpy as jnp
from jax import lax
from jax.experimental import pallas as pl
from jax.experimental.pallas import tpu as pltpu
from jax.experimental.pallas import tpu_sc as plsc

N = 10000
E = 320000
H = 64

NC = 2
NS = 16
GRP = 128

N_PAD = 10240
ROWS_PER_TILE = N_PAD // NS
E_PER_TILE_GRPS = 160
E_PAD = NS * E_PER_TILE_GRPS * GRP

BLK = 1024


_sc_mesh = plsc.VectorSubcoreMesh(core_axis_name="c", subcore_axis_name="s")


@functools.partial(
    pl.kernel,
    out_type=(jax.ShapeDtypeStruct((N_PAD, H), jnp.float32),
              jax.ShapeDtypeStruct((N_PAD, H), jnp.float32)),
    mesh=_sc_mesh,
    compiler_params=pltpu.CompilerParams(use_tc_tiling_on_sc=False),
    scratch_types=[
        pltpu.VMEM_SHARED((N_PAD, H), jnp.float32),
        pltpu.VMEM((E_PER_TILE_GRPS, GRP), jnp.int32),
        pltpu.VMEM((E_PER_TILE_GRPS, GRP), jnp.int32),
        pltpu.VMEM((GRP, H), jnp.float32),
        pltpu.VMEM((GRP, H), jnp.float32),
        pltpu.VMEM((GRP, H), jnp.float32),
        pltpu.VMEM((GRP, H), jnp.float32),
        pltpu.VMEM((GRP, H), jnp.float32),
        pltpu.SemaphoreType.DMA,
        pltpu.SemaphoreType.DMA,
        pltpu.SemaphoreType.DMA,
        pltpu.SemaphoreType.DMA,
        pltpu.SemaphoreType.DMA,
    ],
)
def _sc_dual_segment_sum(yf, yb, src3, dst3, zeros, out_f, out_b,
                         acc, idx_in, idx_out, r0, r1, r2, r3, r4,
                         g0, g1, g2, g3, g4):
    bufs = (r0, r1, r2, r3, r4)
    semg = (g0, g1, g2, g3, g4)
    c = lax.axis_index("c")
    s = lax.axis_index("s")

    acc_slice = acc.at[pl.ds(s * ROWS_PER_TILE, ROWS_PER_TILE)]
    pltpu.async_copy(zeros, acc_slice, g0)

    @pl.when(c == 0)
    def _():
        pltpu.async_copy(src3.at[s], idx_in, g1)
        pltpu.async_copy(dst3.at[s], idx_out, g2)

    @pl.when(c == 1)
    def _():
        pltpu.async_copy(dst3.at[s], idx_in, g1)
        pltpu.async_copy(src3.at[s], idx_out, g2)

    pltpu.make_async_copy(zeros, acc_slice, g0).wait()
    pltpu.make_async_copy(src3.at[s], idx_in, g1).wait()
    pltpu.make_async_copy(src3.at[s], idx_out, g2).wait()

    plsc.subcore_barrier()

    def sweep(table):
        nb = 5
        for b in range(nb - 1):
            pltpu.async_copy(table.at[idx_in.at[b]], bufs[b], semg[b])

        def body(i, carry):
            base = nb * i
            for p in range(nb):
                j = base + p
                q = (p + nb - 1) % nb
                pltpu.async_copy(table.at[idx_in.at[j + nb - 1]], bufs[q],
                                 semg[q])
                pltpu.make_async_copy(table.at[idx_in.at[j]], bufs[p],
                                      semg[p]).wait()
                pltpu.sync_copy(bufs[p], acc.at[idx_out.at[j]], add=True)
            return carry

        lax.fori_loop(0, E_PER_TILE_GRPS // nb - 1, body, 0)

        last = E_PER_TILE_GRPS - nb
        pltpu.async_copy(table.at[idx_in.at[last + nb - 1]],
                         bufs[nb - 1], semg[nb - 1])
        for p in range(nb):
            j = last + p
            pltpu.make_async_copy(table.at[idx_in.at[j]], bufs[p],
                                  semg[p]).wait()
            pltpu.sync_copy(bufs[p], acc.at[idx_out.at[j]], add=True)

    @pl.when(c == 0)
    def _():
        sweep(yf)

    @pl.when(c == 1)
    def _():
        sweep(yb)

    plsc.subcore_barrier()

    @pl.when(c == 0)
    def _():
        pltpu.sync_copy(acc.at[pl.ds(s * ROWS_PER_TILE, ROWS_PER_TILE)],
                        out_f.at[pl.ds(s * ROWS_PER_TILE, ROWS_PER_TILE)])

    @pl.when(c == 1)
    def _():
        pltpu.sync_copy(acc.at[pl.ds(s * ROWS_PER_TILE, ROWS_PER_TILE)],
                        out_b.at[pl.ds(s * ROWS_PER_TILE, ROWS_PER_TILE)])


def _full(shape):
    return pl.BlockSpec(shape, lambda i: (0,) * len(shape))


def _rows(shape):
    return pl.BlockSpec(shape, lambda i: (i,) + (0,) * (len(shape) - 1))


def _stage_a_body(x, wnf, wnb, wrf, wrb, b1f, b1b, wp, bp,
                  y1f, y1b, r1f, r1b, proj):
    xb = x[...]
    y1f[...] = jnp.dot(xb, wnf[...], preferred_element_type=jnp.float32)
    y1b[...] = jnp.dot(xb, wnb[...], preferred_element_type=jnp.float32)
    r1f[...] = jnp.dot(xb, wrf[...], preferred_element_type=jnp.float32) + b1f[...]
    r1b[...] = jnp.dot(xb, wrb[...], preferred_element_type=jnp.float32) + b1b[...]
    proj[...] = jnp.dot(xb, wp[...], preferred_element_type=jnp.float32) + bp[...]


def _stage_a(x, wnf, wnb, wrf, wrb, b1f, b1b, wp, bp):
    o64 = jax.ShapeDtypeStruct((N_PAD, H), jnp.float32)
    o128 = jax.ShapeDtypeStruct((N_PAD, 128), jnp.float32)
    return pl.pallas_call(
        _stage_a_body,
        grid=(N_PAD // BLK,),
        in_specs=[_rows((BLK, 128)), _full((128, H)), _full((128, H)),
                  _full((128, H)), _full((128, H)), _full((1, H)),
                  _full((1, H)), _full((128, 128)), _full((1, 128))],
        out_specs=[_rows((BLK, H)), _rows((BLK, H)), _rows((BLK, H)),
                   _rows((BLK, H)), _rows((BLK, 128))],
        out_shape=[o64, o64, o64, o64, o128],
    )(x, wnf, wnb, wrf, wrb, b1f, b1b, wp, bp)


def _stage_b_body(msgf, msgb, r1f, r1b, wfct, wfcb, bfc,
                  wnf, wnb, wrf, wrb, b2f, b2b,
                  y2f, y2b, r2f, r2b):
    hf = jnp.maximum(r1f[...] + msgf[...], 0.0)
    hb = jnp.maximum(r1b[...] + msgb[...], 0.0)
    h = (jnp.dot(hf, wfct[...], preferred_element_type=jnp.float32)
         + jnp.dot(hb, wfcb[...], preferred_element_type=jnp.float32)
         + bfc[...])
    y2f[...] = jnp.dot(h, wnf[...], preferred_element_type=jnp.float32)
    y2b[...] = jnp.dot(h, wnb[...], preferred_element_type=jnp.float32)
    r2f[...] = jnp.dot(h, wrf[...], preferred_element_type=jnp.float32) + b2f[...]
    r2b[...] = jnp.dot(h, wrb[...], preferred_element_type=jnp.float32) + b2b[...]


def _stage_b(msgf, msgb, r1f, r1b, wfct, wfcb, bfc, wnf, wnb, wrf, wrb,
             b2f, b2b):
    n = msgf.shape[0]
    o64 = jax.ShapeDtypeStruct((n, H), jnp.float32)
    w64 = _full((H, H))
    return pl.pallas_call(
        _stage_b_body,
        grid=(n // BLK,),
        in_specs=[_rows((BLK, H))] * 4 + [w64, w64, _full((1, H)),
                                          w64, w64, w64, w64,
                                          _full((1, H)), _full((1, H))],
        out_specs=[_rows((BLK, H))] * 4,
        out_shape=[o64, o64, o64, o64],
    )(msgf, msgb, r1f, r1b, wfct, wfcb, bfc, wnf, wnb, wrf, wrb, b2f, b2b)


def _stage_c_body(msgf, msgb, r2f, r2b, proj, out):
    of = r2f[...] + msgf[...]
    ob = r2b[...] + msgb[...]
    out[...] = jnp.concatenate([of, ob], axis=-1) + proj[...]


def _stage_c(msgf, msgb, r2f, r2b, proj):
    blk = 1000
    return pl.pallas_call(
        _stage_c_body,
        grid=(N // blk,),
        in_specs=[_rows((blk, H))] * 4 + [_rows((blk, 128))],
        out_specs=_rows((blk, 128)),
        out_shape=jax.ShapeDtypeStruct((N, 128), jnp.float32),
    )(msgf, msgb, r2f, r2b, proj)


def kernel(x, edge_index, W1_root_f, W1_nbr_f, b1_f, W1_root_b, W1_nbr_b,
           b1_b, W_fc, b_fc, W2_root_f, W2_nbr_f, b2_f, W2_root_b, W2_nbr_b,
           b2_b, W_proj, b_proj):
    f32 = jnp.float32

    src = edge_index[0]
    dst = edge_index[1]
    pad = N + (jnp.arange(E_PAD - E, dtype=jnp.int32) % (N_PAD - N))
    src3 = jnp.concatenate([src, pad]).reshape(NS, E_PER_TILE_GRPS, GRP)
    dst3 = jnp.concatenate([dst, pad]).reshape(NS, E_PER_TILE_GRPS, GRP)
    zeros = jnp.zeros((ROWS_PER_TILE, H), f32)

    r1 = lambda b: b.reshape(1, -1)

    y1f, y1b, r1f, r1b, proj = _stage_a(
        x, W1_nbr_f, W1_nbr_b, W1_root_f, W1_root_b,
        r1(b1_f), r1(b1_b), W_proj, r1(b_proj))

    msg1f, msg1b = _sc_dual_segment_sum(y1f, y1b, src3, dst3, zeros)

    y2f, y2b, r2f, r2b = _stage_b(
        msg1f, msg1b, r1f, r1b, W_fc[:H], W_fc[H:], r1(b_fc),
        W2_nbr_f, W2_nbr_b, W2_root_f, W2_root_b, r1(b2_f), r1(b2_b))

    msg2f, msg2b = _sc_dual_segment_sum(y2f, y2b, src3, dst3, zeros)

    return _stage_c(msg2f, msg2b, r2f, r2b, proj)


# speedup vs baseline: 1.0312x; 1.0041x over previous
import functools

import jax
import jax.numpy as jnp
from jax import lax
from jax.experimental import pallas as pl
from jax.experimental.pallas import tpu as pltpu
from jax.experimental.pallas import tpu_sc as plsc

N = 10000
E = 320000
H = 64

NC = 2
NS = 16
GRP = 128

N_PAD = 10240
ROWS_PER_TILE = N_PAD // NS
E_PER_TILE_GRPS = 160
E_PAD = NS * E_PER_TILE_GRPS * GRP

BLK = 1024


_sc_mesh = plsc.VectorSubcoreMesh(core_axis_name="c", subcore_axis_name="s")


@functools.partial(
    pl.kernel,
    out_type=(jax.ShapeDtypeStruct((N_PAD, H), jnp.float32),
              jax.ShapeDtypeStruct((N_PAD, H), jnp.float32)),
    mesh=_sc_mesh,
    compiler_params=pltpu.CompilerParams(use_tc_tiling_on_sc=False),
    scratch_types=[
        pltpu.VMEM_SHARED((N_PAD, H), jnp.float32),
        pltpu.VMEM((E_PER_TILE_GRPS, GRP), jnp.int32),
        pltpu.VMEM((E_PER_TILE_GRPS, GRP), jnp.int32),
        pltpu.VMEM((GRP, H), jnp.float32),
        pltpu.VMEM((GRP, H), jnp.float32),
        pltpu.VMEM((GRP, H), jnp.float32),
        pltpu.VMEM((GRP, H), jnp.float32),
        pltpu.VMEM((GRP, H), jnp.float32),
        pltpu.SemaphoreType.DMA,
        pltpu.SemaphoreType.DMA,
        pltpu.SemaphoreType.DMA,
        pltpu.SemaphoreType.DMA,
        pltpu.SemaphoreType.DMA,
    ],
)
def _sc_dual_segment_sum(yf, yb, src3, dst3, zeros, out_f, out_b,
                         acc, idx_in, idx_out, r0, r1, r2, r3, r4,
                         g0, g1, g2, g3, g4):
    bufs = (r0, r1, r2, r3, r4)
    semg = (g0, g1, g2, g3, g4)
    c = lax.axis_index("c")
    s = lax.axis_index("s")

    nb = 5
    acc_slice = acc.at[pl.ds(s * ROWS_PER_TILE, ROWS_PER_TILE)]

    def prologue(table, gat3, sca3):
        pltpu.async_copy(zeros, acc_slice, g4)
        ci = pltpu.async_copy(gat3.at[s], idx_in, g0)
        co = pltpu.async_copy(sca3.at[s], idx_out, g1)
        ci.wait()
        co.wait()
        for b in range(nb - 1):
            pltpu.async_copy(table.at[idx_in.at[b]], bufs[b], semg[b])

    @pl.when(c == 0)
    def _():
        prologue(yf, src3, dst3)

    @pl.when(c == 1)
    def _():
        prologue(yb, dst3, src3)

    pltpu.make_async_copy(zeros, acc_slice, g4).wait()
    plsc.subcore_barrier()

    def sweep(table):

        def body(i, carry):
            base = nb * i
            for p in range(nb):
                j = base + p
                q = (p + nb - 1) % nb
                pltpu.async_copy(table.at[idx_in.at[j + nb - 1]], bufs[q],
                                 semg[q])
                pltpu.make_async_copy(table.at[idx_in.at[j]], bufs[p],
                                      semg[p]).wait()
                pltpu.sync_copy(bufs[p], acc.at[idx_out.at[j]], add=True)
            return carry

        lax.fori_loop(0, E_PER_TILE_GRPS // nb - 1, body, 0)

        last = E_PER_TILE_GRPS - nb
        pltpu.async_copy(table.at[idx_in.at[last + nb - 1]],
                         bufs[nb - 1], semg[nb - 1])
        for p in range(nb):
            j = last + p
            pltpu.make_async_copy(table.at[idx_in.at[j]], bufs[p],
                                  semg[p]).wait()
            pltpu.sync_copy(bufs[p], acc.at[idx_out.at[j]], add=True)

    @pl.when(c == 0)
    def _():
        sweep(yf)

    @pl.when(c == 1)
    def _():
        sweep(yb)

    plsc.subcore_barrier()

    @pl.when(c == 0)
    def _():
        pltpu.sync_copy(acc.at[pl.ds(s * ROWS_PER_TILE, ROWS_PER_TILE)],
                        out_f.at[pl.ds(s * ROWS_PER_TILE, ROWS_PER_TILE)])

    @pl.when(c == 1)
    def _():
        pltpu.sync_copy(acc.at[pl.ds(s * ROWS_PER_TILE, ROWS_PER_TILE)],
                        out_b.at[pl.ds(s * ROWS_PER_TILE, ROWS_PER_TILE)])


def _full(shape):
    return pl.BlockSpec(shape, lambda i: (0,) * len(shape))


def _rows(shape):
    return pl.BlockSpec(shape, lambda i: (i,) + (0,) * (len(shape) - 1))


def _stage_a_body(x, wnf, wnb, wrf, wrb, b1f, b1b, wp, bp,
                  y1f, y1b, r1f, r1b, proj):
    xb = x[...]
    y1f[...] = jnp.dot(xb, wnf[...], preferred_element_type=jnp.float32)
    y1b[...] = jnp.dot(xb, wnb[...], preferred_element_type=jnp.float32)
    r1f[...] = jnp.dot(xb, wrf[...], preferred_element_type=jnp.float32) + b1f[...]
    r1b[...] = jnp.dot(xb, wrb[...], preferred_element_type=jnp.float32) + b1b[...]
    proj[...] = jnp.dot(xb, wp[...], preferred_element_type=jnp.float32) + bp[...]


def _stage_a(x, wnf, wnb, wrf, wrb, b1f, b1b, wp, bp):
    o64 = jax.ShapeDtypeStruct((N_PAD, H), jnp.float32)
    o128 = jax.ShapeDtypeStruct((N_PAD, 128), jnp.float32)
    return pl.pallas_call(
        _stage_a_body,
        grid=(N_PAD // BLK,),
        in_specs=[_rows((BLK, 128)), _full((128, H)), _full((128, H)),
                  _full((128, H)), _full((128, H)), _full((1, H)),
                  _full((1, H)), _full((128, 128)), _full((1, 128))],
        out_specs=[_rows((BLK, H)), _rows((BLK, H)), _rows((BLK, H)),
                   _rows((BLK, H)), _rows((BLK, 128))],
        out_shape=[o64, o64, o64, o64, o128],
    )(x, wnf, wnb, wrf, wrb, b1f, b1b, wp, bp)


def _stage_b_body(msgf, msgb, r1f, r1b, wfct, wfcb, bfc,
                  wnf, wnb, wrf, wrb, b2f, b2b,
                  y2f, y2b, r2f, r2b):
    hf = jnp.maximum(r1f[...] + msgf[...], 0.0)
    hb = jnp.maximum(r1b[...] + msgb[...], 0.0)
    h = (jnp.dot(hf, wfct[...], preferred_element_type=jnp.float32)
         + jnp.dot(hb, wfcb[...], preferred_element_type=jnp.float32)
         + bfc[...])
    y2f[...] = jnp.dot(h, wnf[...], preferred_element_type=jnp.float32)
    y2b[...] = jnp.dot(h, wnb[...], preferred_element_type=jnp.float32)
    r2f[...] = jnp.dot(h, wrf[...], preferred_element_type=jnp.float32) + b2f[...]
    r2b[...] = jnp.dot(h, wrb[...], preferred_element_type=jnp.float32) + b2b[...]


def _stage_b(msgf, msgb, r1f, r1b, wfct, wfcb, bfc, wnf, wnb, wrf, wrb,
             b2f, b2b):
    n = msgf.shape[0]
    o64 = jax.ShapeDtypeStruct((n, H), jnp.float32)
    w64 = _full((H, H))
    return pl.pallas_call(
        _stage_b_body,
        grid=(n // BLK,),
        in_specs=[_rows((BLK, H))] * 4 + [w64, w64, _full((1, H)),
                                          w64, w64, w64, w64,
                                          _full((1, H)), _full((1, H))],
        out_specs=[_rows((BLK, H))] * 4,
        out_shape=[o64, o64, o64, o64],
    )(msgf, msgb, r1f, r1b, wfct, wfcb, bfc, wnf, wnb, wrf, wrb, b2f, b2b)


def _stage_c_body(msgf, msgb, r2f, r2b, proj, out):
    of = r2f[...] + msgf[...]
    ob = r2b[...] + msgb[...]
    out[...] = jnp.concatenate([of, ob], axis=-1) + proj[...]


def _stage_c(msgf, msgb, r2f, r2b, proj):
    blk = 1000
    return pl.pallas_call(
        _stage_c_body,
        grid=(N // blk,),
        in_specs=[_rows((blk, H))] * 4 + [_rows((blk, 128))],
        out_specs=_rows((blk, 128)),
        out_shape=jax.ShapeDtypeStruct((N, 128), jnp.float32),
    )(msgf, msgb, r2f, r2b, proj)


def kernel(x, edge_index, W1_root_f, W1_nbr_f, b1_f, W1_root_b, W1_nbr_b,
           b1_b, W_fc, b_fc, W2_root_f, W2_nbr_f, b2_f, W2_root_b, W2_nbr_b,
           b2_b, W_proj, b_proj):
    f32 = jnp.float32

    src = edge_index[0]
    dst = edge_index[1]
    pad = N + (jnp.arange(E_PAD - E, dtype=jnp.int32) % (N_PAD - N))
    src3 = jnp.concatenate([src, pad]).reshape(NS, E_PER_TILE_GRPS, GRP)
    dst3 = jnp.concatenate([dst, pad]).reshape(NS, E_PER_TILE_GRPS, GRP)
    zeros = jnp.zeros((ROWS_PER_TILE, H), f32)

    r1 = lambda b: b.reshape(1, -1)

    y1f, y1b, r1f, r1b, proj = _stage_a(
        x, W1_nbr_f, W1_nbr_b, W1_root_f, W1_root_b,
        r1(b1_f), r1(b1_b), W_proj, r1(b_proj))

    msg1f, msg1b = _sc_dual_segment_sum(y1f, y1b, src3, dst3, zeros)

    y2f, y2b, r2f, r2b = _stage_b(
        msg1f, msg1b, r1f, r1b, W_fc[:H], W_fc[H:], r1(b_fc),
        W2_nbr_f, W2_nbr_b, W2_root_f, W2_root_b, r1(b2_f), r1(b2_b))

    msg2f, msg2b = _sc_dual_segment_sum(y2f, y2b, src3, dst3, zeros)

    return _stage_c(msg2f, msg2b, r2f, r2b, proj)


# speedup vs baseline: 1.0472x; 1.0156x over previous
import functools

import jax
import jax.numpy as jnp
from jax import lax
from jax.experimental import pallas as pl
from jax.experimental.pallas import tpu as pltpu
from jax.experimental.pallas import tpu_sc as plsc

N = 10000
E = 320000
H = 64

NC = 2
NS = 16
GRP = 128

N_PAD = 10240
ROWS_PER_TILE = N_PAD // NS
E_PER_TILE_GRPS = 160
E_PAD = NS * E_PER_TILE_GRPS * GRP

BLK = 2048


_sc_mesh = plsc.VectorSubcoreMesh(core_axis_name="c", subcore_axis_name="s")


@functools.partial(
    pl.kernel,
    out_type=(jax.ShapeDtypeStruct((N_PAD, H), jnp.float32),
              jax.ShapeDtypeStruct((N_PAD, H), jnp.float32)),
    mesh=_sc_mesh,
    compiler_params=pltpu.CompilerParams(use_tc_tiling_on_sc=False),
    scratch_types=[
        pltpu.VMEM_SHARED((N_PAD, H), jnp.float32),
        pltpu.VMEM((E_PER_TILE_GRPS, GRP), jnp.int32),
        pltpu.VMEM((E_PER_TILE_GRPS, GRP), jnp.int32),
        pltpu.VMEM((GRP, H), jnp.float32),
        pltpu.VMEM((GRP, H), jnp.float32),
        pltpu.VMEM((GRP, H), jnp.float32),
        pltpu.VMEM((GRP, H), jnp.float32),
        pltpu.VMEM((GRP, H), jnp.float32),
        pltpu.SemaphoreType.DMA,
        pltpu.SemaphoreType.DMA,
        pltpu.SemaphoreType.DMA,
        pltpu.SemaphoreType.DMA,
        pltpu.SemaphoreType.DMA,
    ],
)
def _sc_dual_segment_sum(yf, yb, src3, dst3, zeros, out_f, out_b,
                         acc, idx_in, idx_out, r0, r1, r2, r3, r4,
                         g0, g1, g2, g3, g4):
    bufs = (r0, r1, r2, r3, r4)
    semg = (g0, g1, g2, g3, g4)
    c = lax.axis_index("c")
    s = lax.axis_index("s")

    nb = 5
    acc_slice = acc.at[pl.ds(s * ROWS_PER_TILE, ROWS_PER_TILE)]

    def prologue(table, gat3, sca3):
        pltpu.async_copy(zeros, acc_slice, g4)
        ci = pltpu.async_copy(gat3.at[s], idx_in, g0)
        co = pltpu.async_copy(sca3.at[s], idx_out, g1)
        ci.wait()
        co.wait()
        for b in range(nb - 1):
            pltpu.async_copy(table.at[idx_in.at[b]], bufs[b], semg[b])

    @pl.when(c == 0)
    def _():
        prologue(yf, src3, dst3)

    @pl.when(c == 1)
    def _():
        prologue(yb, dst3, src3)

    pltpu.make_async_copy(zeros, acc_slice, g4).wait()
    plsc.subcore_barrier()

    def sweep(table):

        def body(i, carry):
            base = nb * i
            for p in range(nb):
                j = base + p
                q = (p + nb - 1) % nb
                pltpu.async_copy(table.at[idx_in.at[j + nb - 1]], bufs[q],
                                 semg[q])
                pltpu.make_async_copy(table.at[idx_in.at[j]], bufs[p],
                                      semg[p]).wait()
                pltpu.sync_copy(bufs[p], acc.at[idx_out.at[j]], add=True)
            return carry

        lax.fori_loop(0, E_PER_TILE_GRPS // nb - 1, body, 0)

        last = E_PER_TILE_GRPS - nb
        pltpu.async_copy(table.at[idx_in.at[last + nb - 1]],
                         bufs[nb - 1], semg[nb - 1])
        for p in range(nb):
            j = last + p
            pltpu.make_async_copy(table.at[idx_in.at[j]], bufs[p],
                                  semg[p]).wait()
            pltpu.sync_copy(bufs[p], acc.at[idx_out.at[j]], add=True)

    @pl.when(c == 0)
    def _():
        sweep(yf)

    @pl.when(c == 1)
    def _():
        sweep(yb)

    plsc.subcore_barrier()

    @pl.when(c == 0)
    def _():
        pltpu.sync_copy(acc.at[pl.ds(s * ROWS_PER_TILE, ROWS_PER_TILE)],
                        out_f.at[pl.ds(s * ROWS_PER_TILE, ROWS_PER_TILE)])

    @pl.when(c == 1)
    def _():
        pltpu.sync_copy(acc.at[pl.ds(s * ROWS_PER_TILE, ROWS_PER_TILE)],
                        out_b.at[pl.ds(s * ROWS_PER_TILE, ROWS_PER_TILE)])


def _full(shape):
    return pl.BlockSpec(shape, lambda i: (0,) * len(shape))


def _rows(shape):
    return pl.BlockSpec(shape, lambda i: (i,) + (0,) * (len(shape) - 1))


def _stage_a_body(x, wnf, wnb, wrf, wrb, b1f, b1b, wp, bp,
                  y1f, y1b, r1f, r1b, proj):
    xb = x[...]
    y1f[...] = jnp.dot(xb, wnf[...], preferred_element_type=jnp.float32)
    y1b[...] = jnp.dot(xb, wnb[...], preferred_element_type=jnp.float32)
    r1f[...] = jnp.dot(xb, wrf[...], preferred_element_type=jnp.float32) + b1f[...]
    r1b[...] = jnp.dot(xb, wrb[...], preferred_element_type=jnp.float32) + b1b[...]
    proj[...] = jnp.dot(xb, wp[...], preferred_element_type=jnp.float32) + bp[...]


def _stage_a(x, wnf, wnb, wrf, wrb, b1f, b1b, wp, bp):
    o64 = jax.ShapeDtypeStruct((N_PAD, H), jnp.float32)
    o128 = jax.ShapeDtypeStruct((N_PAD, 128), jnp.float32)
    return pl.pallas_call(
        _stage_a_body,
        grid=(N_PAD // BLK,),
        in_specs=[_rows((BLK, 128)), _full((128, H)), _full((128, H)),
                  _full((128, H)), _full((128, H)), _full((1, H)),
                  _full((1, H)), _full((128, 128)), _full((1, 128))],
        out_specs=[_rows((BLK, H)), _rows((BLK, H)), _rows((BLK, H)),
                   _rows((BLK, H)), _rows((BLK, 128))],
        out_shape=[o64, o64, o64, o64, o128],
    )(x, wnf, wnb, wrf, wrb, b1f, b1b, wp, bp)


def _stage_b_body(msgf, msgb, r1f, r1b, wfct, wfcb, bfc,
                  wnf, wnb, wrf, wrb, b2f, b2b,
                  y2f, y2b, r2f, r2b):
    hf = jnp.maximum(r1f[...] + msgf[...], 0.0)
    hb = jnp.maximum(r1b[...] + msgb[...], 0.0)
    h = (jnp.dot(hf, wfct[...], preferred_element_type=jnp.float32)
         + jnp.dot(hb, wfcb[...], preferred_element_type=jnp.float32)
         + bfc[...])
    y2f[...] = jnp.dot(h, wnf[...], preferred_element_type=jnp.float32)
    y2b[...] = jnp.dot(h, wnb[...], preferred_element_type=jnp.float32)
    r2f[...] = jnp.dot(h, wrf[...], preferred_element_type=jnp.float32) + b2f[...]
    r2b[...] = jnp.dot(h, wrb[...], preferred_element_type=jnp.float32) + b2b[...]


def _stage_b(msgf, msgb, r1f, r1b, wfct, wfcb, bfc, wnf, wnb, wrf, wrb,
             b2f, b2b):
    n = msgf.shape[0]
    o64 = jax.ShapeDtypeStruct((n, H), jnp.float32)
    w64 = _full((H, H))
    return pl.pallas_call(
        _stage_b_body,
        grid=(n // BLK,),
        in_specs=[_rows((BLK, H))] * 4 + [w64, w64, _full((1, H)),
                                          w64, w64, w64, w64,
                                          _full((1, H)), _full((1, H))],
        out_specs=[_rows((BLK, H))] * 4,
        out_shape=[o64, o64, o64, o64],
    )(msgf, msgb, r1f, r1b, wfct, wfcb, bfc, wnf, wnb, wrf, wrb, b2f, b2b)


def _stage_c_body(msgf, msgb, r2f, r2b, proj, out):
    of = r2f[...] + msgf[...]
    ob = r2b[...] + msgb[...]
    out[...] = jnp.concatenate([of, ob], axis=-1) + proj[...]


def _stage_c(msgf, msgb, r2f, r2b, proj):
    blk = 1000
    return pl.pallas_call(
        _stage_c_body,
        grid=(N // blk,),
        in_specs=[_rows((blk, H))] * 4 + [_rows((blk, 128))],
        out_specs=_rows((blk, 128)),
        out_shape=jax.ShapeDtypeStruct((N, 128), jnp.float32),
    )(msgf, msgb, r2f, r2b, proj)


def kernel(x, edge_index, W1_root_f, W1_nbr_f, b1_f, W1_root_b, W1_nbr_b,
           b1_b, W_fc, b_fc, W2_root_f, W2_nbr_f, b2_f, W2_root_b, W2_nbr_b,
           b2_b, W_proj, b_proj):
    f32 = jnp.float32

    src = edge_index[0]
    dst = edge_index[1]
    pad = N + (jnp.arange(E_PAD - E, dtype=jnp.int32) % (N_PAD - N))
    src3 = jnp.concatenate([src, pad]).reshape(NS, E_PER_TILE_GRPS, GRP)
    dst3 = jnp.concatenate([dst, pad]).reshape(NS, E_PER_TILE_GRPS, GRP)
    zeros = jnp.zeros((ROWS_PER_TILE, H), f32)

    r1 = lambda b: b.reshape(1, -1)

    y1f, y1b, r1f, r1b, proj = _stage_a(
        x, W1_nbr_f, W1_nbr_b, W1_root_f, W1_root_b,
        r1(b1_f), r1(b1_b), W_proj, r1(b_proj))

    msg1f, msg1b = _sc_dual_segment_sum(y1f, y1b, src3, dst3, zeros)

    y2f, y2b, r2f, r2b = _stage_b(
        msg1f, msg1b, r1f, r1b, W_fc[:H], W_fc[H:], r1(b_fc),
        W2_nbr_f, W2_nbr_b, W2_root_f, W2_root_b, r1(b2_f), r1(b2_b))

    msg2f, msg2b = _sc_dual_segment_sum(y2f, y2b, src3, dst3, zeros)

    return _stage_c(msg2f, msg2b, r2f, r2b, proj)


# speedup vs baseline: 1.0554x; 1.0078x over previous
import functools

import jax
import jax.numpy as jnp
from jax import lax
from jax.experimental import pallas as pl
from jax.experimental.pallas import tpu as pltpu
from jax.experimental.pallas import tpu_sc as plsc

N = 10000
E = 320000
H = 64

NC = 2
NS = 16
GRP = 128

N_PAD = 10240
ROWS_PER_TILE = N_PAD // NS
E_PER_TILE_GRPS = 160
E_PAD = NS * E_PER_TILE_GRPS * GRP

BLK = 5120


_sc_mesh = plsc.VectorSubcoreMesh(core_axis_name="c", subcore_axis_name="s")


@functools.partial(
    pl.kernel,
    out_type=(jax.ShapeDtypeStruct((N_PAD, H), jnp.float32),
              jax.ShapeDtypeStruct((N_PAD, H), jnp.float32)),
    mesh=_sc_mesh,
    compiler_params=pltpu.CompilerParams(use_tc_tiling_on_sc=False),
    scratch_types=[
        pltpu.VMEM_SHARED((N_PAD, H), jnp.float32),
        pltpu.VMEM((E_PER_TILE_GRPS, GRP), jnp.int32),
        pltpu.VMEM((E_PER_TILE_GRPS, GRP), jnp.int32),
        pltpu.VMEM((GRP, H), jnp.float32),
        pltpu.VMEM((GRP, H), jnp.float32),
        pltpu.VMEM((GRP, H), jnp.float32),
        pltpu.VMEM((GRP, H), jnp.float32),
        pltpu.VMEM((GRP, H), jnp.float32),
        pltpu.SemaphoreType.DMA,
        pltpu.SemaphoreType.DMA,
        pltpu.SemaphoreType.DMA,
        pltpu.SemaphoreType.DMA,
        pltpu.SemaphoreType.DMA,
    ],
)
def _sc_dual_segment_sum(yf, yb, src3, dst3, zeros, out_f, out_b,
                         acc, idx_in, idx_out, r0, r1, r2, r3, r4,
                         g0, g1, g2, g3, g4):
    bufs = (r0, r1, r2, r3, r4)
    semg = (g0, g1, g2, g3, g4)
    c = lax.axis_index("c")
    s = lax.axis_index("s")

    nb = 5
    acc_slice = acc.at[pl.ds(s * ROWS_PER_TILE, ROWS_PER_TILE)]

    def prologue(table, gat3, sca3):
        pltpu.async_copy(zeros, acc_slice, g4)
        ci = pltpu.async_copy(gat3.at[s], idx_in, g0)
        co = pltpu.async_copy(sca3.at[s], idx_out, g1)
        ci.wait()
        co.wait()
        for b in range(nb - 1):
            pltpu.async_copy(table.at[idx_in.at[b]], bufs[b], semg[b])

    @pl.when(c == 0)
    def _():
        prologue(yf, src3, dst3)

    @pl.when(c == 1)
    def _():
        prologue(yb, dst3, src3)

    pltpu.make_async_copy(zeros, acc_slice, g4).wait()
    plsc.subcore_barrier()

    def sweep(table):

        def body(i, carry):
            base = nb * i
            for p in range(nb):
                j = base + p
                q = (p + nb - 1) % nb
                pltpu.async_copy(table.at[idx_in.at[j + nb - 1]], bufs[q],
                                 semg[q])
                pltpu.make_async_copy(table.at[idx_in.at[j]], bufs[p],
                                      semg[p]).wait()
                pltpu.sync_copy(bufs[p], acc.at[idx_out.at[j]], add=True)
            return carry

        lax.fori_loop(0, E_PER_TILE_GRPS // nb - 1, body, 0)

        last = E_PER_TILE_GRPS - nb
        pltpu.async_copy(table.at[idx_in.at[last + nb - 1]],
                         bufs[nb - 1], semg[nb - 1])
        for p in range(nb):
            j = last + p
            pltpu.make_async_copy(table.at[idx_in.at[j]], bufs[p],
                                  semg[p]).wait()
            pltpu.sync_copy(bufs[p], acc.at[idx_out.at[j]], add=True)

    @pl.when(c == 0)
    def _():
        sweep(yf)

    @pl.when(c == 1)
    def _():
        sweep(yb)

    plsc.subcore_barrier()

    @pl.when(c == 0)
    def _():
        pltpu.sync_copy(acc.at[pl.ds(s * ROWS_PER_TILE, ROWS_PER_TILE)],
                        out_f.at[pl.ds(s * ROWS_PER_TILE, ROWS_PER_TILE)])

    @pl.when(c == 1)
    def _():
        pltpu.sync_copy(acc.at[pl.ds(s * ROWS_PER_TILE, ROWS_PER_TILE)],
                        out_b.at[pl.ds(s * ROWS_PER_TILE, ROWS_PER_TILE)])


def _full(shape):
    return pl.BlockSpec(shape, lambda i: (0,) * len(shape))


def _rows(shape):
    return pl.BlockSpec(shape, lambda i: (i,) + (0,) * (len(shape) - 1))


def _stage_a_body(x, wnf, wnb, wrf, wrb, b1f, b1b, wp, bp,
                  y1f, y1b, r1f, r1b, proj):
    xb = x[...]
    y1f[...] = jnp.dot(xb, wnf[...], preferred_element_type=jnp.float32)
    y1b[...] = jnp.dot(xb, wnb[...], preferred_element_type=jnp.float32)
    r1f[...] = jnp.dot(xb, wrf[...], preferred_element_type=jnp.float32) + b1f[...]
    r1b[...] = jnp.dot(xb, wrb[...], preferred_element_type=jnp.float32) + b1b[...]
    proj[...] = jnp.dot(xb, wp[...], preferred_element_type=jnp.float32) + bp[...]


def _stage_a(x, wnf, wnb, wrf, wrb, b1f, b1b, wp, bp):
    o64 = jax.ShapeDtypeStruct((N_PAD, H), jnp.float32)
    o128 = jax.ShapeDtypeStruct((N_PAD, 128), jnp.float32)
    return pl.pallas_call(
        _stage_a_body,
        grid=(N_PAD // BLK,),
        in_specs=[_rows((BLK, 128)), _full((128, H)), _full((128, H)),
                  _full((128, H)), _full((128, H)), _full((1, H)),
                  _full((1, H)), _full((128, 128)), _full((1, 128))],
        out_specs=[_rows((BLK, H)), _rows((BLK, H)), _rows((BLK, H)),
                   _rows((BLK, H)), _rows((BLK, 128))],
        out_shape=[o64, o64, o64, o64, o128],
    )(x, wnf, wnb, wrf, wrb, b1f, b1b, wp, bp)


def _stage_b_body(msgf, msgb, r1f, r1b, wfct, wfcb, bfc,
                  wnf, wnb, wrf, wrb, b2f, b2b,
                  y2f, y2b, r2f, r2b):
    hf = jnp.maximum(r1f[...] + msgf[...], 0.0)
    hb = jnp.maximum(r1b[...] + msgb[...], 0.0)
    h = (jnp.dot(hf, wfct[...], preferred_element_type=jnp.float32)
         + jnp.dot(hb, wfcb[...], preferred_element_type=jnp.float32)
         + bfc[...])
    y2f[...] = jnp.dot(h, wnf[...], preferred_element_type=jnp.float32)
    y2b[...] = jnp.dot(h, wnb[...], preferred_element_type=jnp.float32)
    r2f[...] = jnp.dot(h, wrf[...], preferred_element_type=jnp.float32) + b2f[...]
    r2b[...] = jnp.dot(h, wrb[...], preferred_element_type=jnp.float32) + b2b[...]


def _stage_b(msgf, msgb, r1f, r1b, wfct, wfcb, bfc, wnf, wnb, wrf, wrb,
             b2f, b2b):
    n = msgf.shape[0]
    o64 = jax.ShapeDtypeStruct((n, H), jnp.float32)
    w64 = _full((H, H))
    return pl.pallas_call(
        _stage_b_body,
        grid=(n // BLK,),
        in_specs=[_rows((BLK, H))] * 4 + [w64, w64, _full((1, H)),
                                          w64, w64, w64, w64,
                                          _full((1, H)), _full((1, H))],
        out_specs=[_rows((BLK, H))] * 4,
        out_shape=[o64, o64, o64, o64],
    )(msgf, msgb, r1f, r1b, wfct, wfcb, bfc, wnf, wnb, wrf, wrb, b2f, b2b)


def _stage_c_body(msgf, msgb, r2f, r2b, proj, out):
    of = r2f[...] + msgf[...]
    ob = r2b[...] + msgb[...]
    out[...] = jnp.concatenate([of, ob], axis=-1) + proj[...]


def _stage_c(msgf, msgb, r2f, r2b, proj):
    blk = 1000
    return pl.pallas_call(
        _stage_c_body,
        grid=(N // blk,),
        in_specs=[_rows((blk, H))] * 4 + [_rows((blk, 128))],
        out_specs=_rows((blk, 128)),
        out_shape=jax.ShapeDtypeStruct((N, 128), jnp.float32),
    )(msgf, msgb, r2f, r2b, proj)


def kernel(x, edge_index, W1_root_f, W1_nbr_f, b1_f, W1_root_b, W1_nbr_b,
           b1_b, W_fc, b_fc, W2_root_f, W2_nbr_f, b2_f, W2_root_b, W2_nbr_b,
           b2_b, W_proj, b_proj):
    f32 = jnp.float32

    src = edge_index[0]
    dst = edge_index[1]
    pad = N + (jnp.arange(E_PAD - E, dtype=jnp.int32) % (N_PAD - N))
    src3 = jnp.concatenate([src, pad]).reshape(NS, E_PER_TILE_GRPS, GRP)
    dst3 = jnp.concatenate([dst, pad]).reshape(NS, E_PER_TILE_GRPS, GRP)
    zeros = jnp.zeros((ROWS_PER_TILE, H), f32)

    r1 = lambda b: b.reshape(1, -1)

    y1f, y1b, r1f, r1b, proj = _stage_a(
        x, W1_nbr_f, W1_nbr_b, W1_root_f, W1_root_b,
        r1(b1_f), r1(b1_b), W_proj, r1(b_proj))

    msg1f, msg1b = _sc_dual_segment_sum(y1f, y1b, src3, dst3, zeros)

    y2f, y2b, r2f, r2b = _stage_b(
        msg1f, msg1b, r1f, r1b, W_fc[:H], W_fc[H:], r1(b_fc),
        W2_nbr_f, W2_nbr_b, W2_root_f, W2_root_b, r1(b2_f), r1(b2_b))

    msg2f, msg2b = _sc_dual_segment_sum(y2f, y2b, src3, dst3, zeros)

    return _stage_c(msg2f, msg2b, r2f, r2b, proj)
